# Initial kernel scaffold; baseline (speedup 1.0000x reference)
import jax
import jax.numpy as jnp
from jax.experimental import pallas as pl


def kernel(features, edge_index, W_enc, W_layers, points):
    raise NotImplementedError("write your pallas kernel here")


# speedup vs baseline: 5.0665x; 5.0665x over previous
import functools

import jax
import jax.numpy as jnp
from jax import lax
from jax.experimental import pallas as pl
from jax.experimental.pallas import tpu as pltpu
from jax.experimental.pallas import tpu_sc as plsc

N = 10000
E = 320000
IN_DIM = 128
D = 129
DP = 144
NP = 10240
NCLS = 70
T = 4
N_LAYERS = 2
STEP = 0.1
VTH = 1.0

NCORES = 2
NSUB = 16
NW = NCORES * NSUB
EPW = E // NW
K = 80
NCHUNK = EPW // K
ROWS_PER_SUB = NP // NSUB
ZR = 128

_mesh = plsc.VectorSubcoreMesh(core_axis_name="c", subcore_axis_name="s")


def _msgpass_body(x_hbm, src_hbm, dst_hbm, out_hbm,
                  src_c, dst_c, rows_v, acc_sh, gsem):
    c = lax.axis_index("c")
    s = lax.axis_index("s")
    wid = s * NCORES + c

    def zb(i, _):
        rows_v[i // (DP // 16), pl.ds((i % (DP // 16)) * 16, 16)] = (
            jnp.zeros((16,), jnp.float32))
        return 0
    lax.fori_loop(0, K * (DP // 16), zb, 0)

    def zcp(j, _):
        pltpu.sync_copy(
            rows_v, acc_sh.at[pl.ds(s * ROWS_PER_SUB + j * K, K)])
        return 0
    lax.fori_loop(0, ROWS_PER_SUB // K, zcp, 0)

    plsc.subcore_barrier()

    def body(j, _):
        pltpu.sync_copy(src_hbm.at[wid, j], src_c)
        pltpu.sync_copy(dst_hbm.at[wid, j], dst_c)
        pltpu.async_copy(x_hbm.at[src_c], rows_v, gsem).wait()
        pltpu.sync_copy(rows_v, acc_sh.at[dst_c], add=True)
        return 0
    lax.fori_loop(0, NCHUNK, body, 0)

    plsc.subcore_barrier()
    pltpu.sync_copy(
        acc_sh.at[pl.ds(s * ROWS_PER_SUB, ROWS_PER_SUB)],
        out_hbm.at[c, pl.ds(s * ROWS_PER_SUB, ROWS_PER_SUB)])


_msgpass = functools.partial(
    pl.kernel,
    out_type=jax.ShapeDtypeStruct((NCORES, NP, DP), jnp.float32),
    mesh=_mesh,
    scratch_types=[
        pltpu.VMEM((K,), jnp.int32),
        pltpu.VMEM((K,), jnp.int32),
        pltpu.VMEM((K, DP), jnp.float32),
        pltpu.VMEM_SHARED((NP, DP), jnp.float32),
        pltpu.SemaphoreType.DMA,
    ],
    compiler_params=pltpu.CompilerParams(use_tc_tiling_on_sc=False),
)(_msgpass_body)


def _col_ids(shape):
    return lax.broadcasted_iota(jnp.int32, shape, len(shape) - 1)


def _if_dynamics(agg):
    v = jnp.zeros_like(agg)
    ssum = jnp.zeros_like(agg)
    for _ in range(T):
        v = v + agg
        spk = (v >= VTH).astype(jnp.float32)
        v = v - spk * VTH
        ssum = ssum + spk
    return ssum


def _projx(t):
    ids = _col_ids(t.shape)
    rest = jnp.where((ids >= 1) & (ids < D), t, 0.0)
    rss = jnp.sum(rest * rest, axis=-1, keepdims=True)
    x0 = jnp.sqrt(1.0 + rss)
    return jnp.where(ids == 0, x0, rest)


NB = 8
BR = NP // NB


def _encode_body(f_ref, w_ref, out_ref):
    mm = jnp.dot(f_ref[...], w_ref[...], preferred_element_type=jnp.float32)
    out_ref[...] = mm + (_col_ids(mm.shape) == D).astype(jnp.float32)


_encode = pl.pallas_call(
    _encode_body,
    grid=(NB,),
    in_specs=[
        pl.BlockSpec((BR, IN_DIM), lambda i: (i, 0)),
        pl.BlockSpec((IN_DIM, DP), lambda i: (0, 0)),
    ],
    out_specs=pl.BlockSpec((BR, DP), lambda i: (i, 0)),
    out_shape=jax.ShapeDtypeStruct((NP, DP), jnp.float32),
)


def _stage0_body(p_ref, w_ref, x_ref, z_ref, dinv_ref):
    ssum = p_ref[0] + p_ref[1]
    deg = ssum[:, D:D + 1]
    dinv = 1.0 / jnp.maximum(deg, 1.0)
    agg = ssum * dinv
    xs = _if_dynamics(agg)
    xs = jnp.where(_col_ids(xs.shape) < D, xs, 0.0)
    x_ref[...] = jnp.dot(xs, w_ref[...], preferred_element_type=jnp.float32)
    z_ref[...] = _projx(xs * STEP)
    dinv_ref[...] = dinv


_stage0 = pl.pallas_call(
    _stage0_body,
    grid=(NB,),
    in_specs=[
        pl.BlockSpec((2, BR, DP), lambda i: (0, i, 0)),
        pl.BlockSpec((DP, DP), lambda i: (0, 0)),
    ],
    out_specs=(
        pl.BlockSpec((BR, DP), lambda i: (i, 0)),
        pl.BlockSpec((BR, DP), lambda i: (i, 0)),
        pl.BlockSpec((BR, 1), lambda i: (i, 0)),
    ),
    out_shape=(
        jax.ShapeDtypeStruct((NP, DP), jnp.float32),
        jax.ShapeDtypeStruct((NP, DP), jnp.float32),
        jax.ShapeDtypeStruct((NP, 1), jnp.float32),
    ),
)


def _stage1_body(p_ref, w_ref, z_in_ref, dinv_ref, x_ref, z_ref):
    ssum = p_ref[0] + p_ref[1]
    agg = ssum * dinv_ref[...]
    xs = _if_dynamics(agg)
    xs = jnp.where(_col_ids(xs.shape) < D, xs, 0.0)
    x_ref[...] = jnp.dot(xs, w_ref[...], preferred_element_type=jnp.float32)
    z_ref[...] = _projx(z_in_ref[...] + xs * STEP)


_stage1 = pl.pallas_call(
    _stage1_body,
    grid=(NB,),
    in_specs=[
        pl.BlockSpec((2, BR, DP), lambda i: (0, i, 0)),
        pl.BlockSpec((DP, DP), lambda i: (0, 0)),
        pl.BlockSpec((BR, DP), lambda i: (i, 0)),
        pl.BlockSpec((BR, 1), lambda i: (i, 0)),
    ],
    out_specs=(
        pl.BlockSpec((BR, DP), lambda i: (i, 0)),
        pl.BlockSpec((BR, DP), lambda i: (i, 0)),
    ),
    out_shape=(
        jax.ShapeDtypeStruct((NP, DP), jnp.float32),
        jax.ShapeDtypeStruct((NP, DP), jnp.float32),
    ),
)


def _final_body(p_ref, z_in_ref, dinv_ref, pts_ref, out_ref):
    ssum = p_ref[0] + p_ref[1]
    agg = ssum * dinv_ref[...]
    xs = _if_dynamics(agg)
    xs = jnp.where(_col_ids(xs.shape) < D, xs, 0.0)
    z = _projx(z_in_ref[...] + xs * STEP)
    q = _projx(pts_ref[...])
    ids = _col_ids(z.shape)
    zneg = jnp.where(ids == 0, -z, z)
    inner = lax.dot_general(zneg, q, (((1,), (1,)), ((), ())),
                            preferred_element_type=jnp.float32)
    c = jnp.maximum(-inner, 1.0 + 1e-5)
    out_ref[...] = -jnp.log(c + jnp.sqrt((c - 1.0) * (c + 1.0)))


_final = pl.pallas_call(
    _final_body,
    grid=(NB,),
    in_specs=[
        pl.BlockSpec((2, BR, DP), lambda i: (0, i, 0)),
        pl.BlockSpec((BR, DP), lambda i: (i, 0)),
        pl.BlockSpec((BR, 1), lambda i: (i, 0)),
        pl.BlockSpec((128, DP), lambda i: (0, 0)),
    ],
    out_specs=pl.BlockSpec((BR, 128), lambda i: (i, 0)),
    out_shape=jax.ShapeDtypeStruct((NP, 128), jnp.float32),
)


def kernel(features, edge_index, W_enc, W_layers, points):
    src = edge_index[0].reshape(NW, NCHUNK, K)
    dst = edge_index[1].reshape(NW, NCHUNK, K)
    f_p = jnp.zeros((NP, IN_DIM), jnp.float32).at[:N].set(features)
    we_p = jnp.zeros((IN_DIM, DP), jnp.float32).at[:, :D].set(W_enc)
    wl_p = jnp.zeros((N_LAYERS, DP, DP), jnp.float32).at[:, :D, :D].set(W_layers)
    pts_p = jnp.zeros((128, DP), jnp.float32).at[:NCLS, :D].set(points)

    x0 = _encode(f_p, we_p)
    p0 = _msgpass(x0, src, dst)
    x1, z0, dinv = _stage0(p0, wl_p[0])
    p1 = _msgpass(x1, src, dst)
    x2, z1 = _stage1(p1, wl_p[1], z0, dinv)
    p2 = _msgpass(x2, src, dst)
    return _final(p2, z1, dinv, pts_p)[:N, :NCLS]


# speedup vs baseline: 8.4499x; 1.6678x over previous
import functools

import jax
import jax.numpy as jnp
from jax import lax
from jax.experimental import pallas as pl
from jax.experimental.pallas import tpu as pltpu
from jax.experimental.pallas import tpu_sc as plsc

N = 10000
E = 320000
IN_DIM = 128
D = 129
DP = 144
NP = 10240
NCLS = 70
T = 4
N_LAYERS = 2
STEP = 0.1
VTH = 1.0

NCORES = 2
NSUB = 16
NW = NCORES * NSUB
EPW = E // NW
K = 40
NCHUNK = EPW // K
ROWS_PER_SUB = NP // NSUB

_mesh = plsc.VectorSubcoreMesh(core_axis_name="c", subcore_axis_name="s")


def _msgpass_body(x_hbm, src_hbm, dst_hbm, out_hbm,
                  src_s, dst_s, rows_a, rows_b, acc_sh, sem_a, sem_b):
    c = lax.axis_index("c")
    s = lax.axis_index("s")
    wid = s * NCORES + c

    def zb(i, _):
        rows_a[i // (DP // 16), pl.ds((i % (DP // 16)) * 16, 16)] = (
            jnp.zeros((16,), jnp.float32))
        return 0
    lax.fori_loop(0, K * (DP // 16), zb, 0)

    def zcp(j, _):
        pltpu.sync_copy(
            rows_a, acc_sh.at[pl.ds(s * ROWS_PER_SUB + j * K, K)])
        return 0
    lax.fori_loop(0, ROWS_PER_SUB // K, zcp, 0)

    pltpu.sync_copy(src_hbm.at[wid], src_s)
    pltpu.sync_copy(dst_hbm.at[wid], dst_s)

    plsc.subcore_barrier()

    pltpu.async_copy(x_hbm.at[src_s.at[0]], rows_a, sem_a)

    def body(i, _):
        j = 2 * i
        pltpu.async_copy(x_hbm.at[src_s.at[j + 1]], rows_b, sem_b)
        pltpu.make_async_copy(x_hbm.at[src_s.at[j]], rows_a, sem_a).wait()
        pltpu.sync_copy(rows_a, acc_sh.at[dst_s.at[j]], add=True)

        @pl.when(i < NCHUNK // 2 - 1)
        def _():
            pltpu.async_copy(x_hbm.at[src_s.at[j + 2]], rows_a, sem_a)
        pltpu.make_async_copy(x_hbm.at[src_s.at[j + 1]], rows_b, sem_b).wait()
        pltpu.sync_copy(rows_b, acc_sh.at[dst_s.at[j + 1]], add=True)
        return 0
    lax.fori_loop(0, NCHUNK // 2, body, 0)

    plsc.subcore_barrier()
    pltpu.sync_copy(
        acc_sh.at[pl.ds(s * ROWS_PER_SUB, ROWS_PER_SUB)],
        out_hbm.at[c, pl.ds(s * ROWS_PER_SUB, ROWS_PER_SUB)])


_msgpass = functools.partial(
    pl.kernel,
    out_type=jax.ShapeDtypeStruct((NCORES, NP, DP), jnp.float32),
    mesh=_mesh,
    scratch_types=[
        pltpu.VMEM((NCHUNK, K), jnp.int32),
        pltpu.VMEM((NCHUNK, K), jnp.int32),
        pltpu.VMEM((K, DP), jnp.float32),
        pltpu.VMEM((K, DP), jnp.float32),
        pltpu.VMEM_SHARED((NP, DP), jnp.float32),
        pltpu.SemaphoreType.DMA,
        pltpu.SemaphoreType.DMA,
    ],
    compiler_params=pltpu.CompilerParams(use_tc_tiling_on_sc=False),
)(_msgpass_body)


def _col_ids(shape):
    return lax.broadcasted_iota(jnp.int32, shape, len(shape) - 1)


def _if_dynamics(agg):
    v = jnp.zeros_like(agg)
    ssum = jnp.zeros_like(agg)
    for _ in range(T):
        v = v + agg
        spk = (v >= VTH).astype(jnp.float32)
        v = v - spk * VTH
        ssum = ssum + spk
    return ssum


def _projx(t):
    ids = _col_ids(t.shape)
    rest = jnp.where((ids >= 1) & (ids < D), t, 0.0)
    rss = jnp.sum(rest * rest, axis=-1, keepdims=True)
    x0 = jnp.sqrt(1.0 + rss)
    return jnp.where(ids == 0, x0, rest)


NB = 8
BR = NP // NB


def _encode_body(f_ref, w_ref, out_ref):
    mm = jnp.dot(f_ref[...], w_ref[...], preferred_element_type=jnp.float32)
    out_ref[...] = mm + (_col_ids(mm.shape) == D).astype(jnp.float32)


_encode = pl.pallas_call(
    _encode_body,
    grid=(NB,),
    in_specs=[
        pl.BlockSpec((BR, IN_DIM), lambda i: (i, 0)),
        pl.BlockSpec((IN_DIM, DP), lambda i: (0, 0)),
    ],
    out_specs=pl.BlockSpec((BR, DP), lambda i: (i, 0)),
    out_shape=jax.ShapeDtypeStruct((NP, DP), jnp.float32),
)


def _stage0_body(p_ref, w_ref, x_ref, z_ref, dinv_ref):
    ssum = p_ref[0] + p_ref[1]
    deg = ssum[:, D:D + 1]
    dinv = 1.0 / jnp.maximum(deg, 1.0)
    agg = ssum * dinv
    xs = _if_dynamics(agg)
    xs = jnp.where(_col_ids(xs.shape) < D, xs, 0.0)
    x_ref[...] = jnp.dot(xs, w_ref[...], preferred_element_type=jnp.float32)
    z_ref[...] = _projx(xs * STEP)
    dinv_ref[...] = dinv


_stage0 = pl.pallas_call(
    _stage0_body,
    grid=(NB,),
    in_specs=[
        pl.BlockSpec((2, BR, DP), lambda i: (0, i, 0)),
        pl.BlockSpec((DP, DP), lambda i: (0, 0)),
    ],
    out_specs=(
        pl.BlockSpec((BR, DP), lambda i: (i, 0)),
        pl.BlockSpec((BR, DP), lambda i: (i, 0)),
        pl.BlockSpec((BR, 1), lambda i: (i, 0)),
    ),
    out_shape=(
        jax.ShapeDtypeStruct((NP, DP), jnp.float32),
        jax.ShapeDtypeStruct((NP, DP), jnp.float32),
        jax.ShapeDtypeStruct((NP, 1), jnp.float32),
    ),
)


def _stage1_body(p_ref, w_ref, z_in_ref, dinv_ref, x_ref, z_ref):
    ssum = p_ref[0] + p_ref[1]
    agg = ssum * dinv_ref[...]
    xs = _if_dynamics(agg)
    xs = jnp.where(_col_ids(xs.shape) < D, xs, 0.0)
    x_ref[...] = jnp.dot(xs, w_ref[...], preferred_element_type=jnp.float32)
    z_ref[...] = _projx(z_in_ref[...] + xs * STEP)


_stage1 = pl.pallas_call(
    _stage1_body,
    grid=(NB,),
    in_specs=[
        pl.BlockSpec((2, BR, DP), lambda i: (0, i, 0)),
        pl.BlockSpec((DP, DP), lambda i: (0, 0)),
        pl.BlockSpec((BR, DP), lambda i: (i, 0)),
        pl.BlockSpec((BR, 1), lambda i: (i, 0)),
    ],
    out_specs=(
        pl.BlockSpec((BR, DP), lambda i: (i, 0)),
        pl.BlockSpec((BR, DP), lambda i: (i, 0)),
    ),
    out_shape=(
        jax.ShapeDtypeStruct((NP, DP), jnp.float32),
        jax.ShapeDtypeStruct((NP, DP), jnp.float32),
    ),
)


def _final_body(p_ref, z_in_ref, dinv_ref, pts_ref, out_ref):
    ssum = p_ref[0] + p_ref[1]
    agg = ssum * dinv_ref[...]
    xs = _if_dynamics(agg)
    xs = jnp.where(_col_ids(xs.shape) < D, xs, 0.0)
    z = _projx(z_in_ref[...] + xs * STEP)
    q = _projx(pts_ref[...])
    ids = _col_ids(z.shape)
    zneg = jnp.where(ids == 0, -z, z)
    inner = lax.dot_general(zneg, q, (((1,), (1,)), ((), ())),
                            preferred_element_type=jnp.float32)
    c = jnp.maximum(-inner, 1.0 + 1e-5)
    out_ref[...] = -jnp.log(c + jnp.sqrt((c - 1.0) * (c + 1.0)))


_final = pl.pallas_call(
    _final_body,
    grid=(NB,),
    in_specs=[
        pl.BlockSpec((2, BR, DP), lambda i: (0, i, 0)),
        pl.BlockSpec((BR, DP), lambda i: (i, 0)),
        pl.BlockSpec((BR, 1), lambda i: (i, 0)),
        pl.BlockSpec((128, DP), lambda i: (0, 0)),
    ],
    out_specs=pl.BlockSpec((BR, 128), lambda i: (i, 0)),
    out_shape=jax.ShapeDtypeStruct((NP, 128), jnp.float32),
)


def kernel(features, edge_index, W_enc, W_layers, points):
    src = edge_index[0].reshape(NW, NCHUNK, K)
    dst = edge_index[1].reshape(NW, NCHUNK, K)
    f_p = jnp.zeros((NP, IN_DIM), jnp.float32).at[:N].set(features)
    we_p = jnp.zeros((IN_DIM, DP), jnp.float32).at[:, :D].set(W_enc)
    wl_p = jnp.zeros((N_LAYERS, DP, DP), jnp.float32).at[:, :D, :D].set(W_layers)
    pts_p = jnp.zeros((128, DP), jnp.float32).at[:NCLS, :D].set(points)

    x0 = _encode(f_p, we_p)
    p0 = _msgpass(x0, src, dst)
    x1, z0, dinv = _stage0(p0, wl_p[0])
    p1 = _msgpass(x1, src, dst)
    x2, z1 = _stage1(p1, wl_p[1], z0, dinv)
    p2 = _msgpass(x2, src, dst)
    return _final(p2, z1, dinv, pts_p)[:N, :NCLS]


# speedup vs baseline: 9.8109x; 1.1611x over previous
import functools

import jax
import jax.numpy as jnp
from jax import lax
from jax.experimental import pallas as pl
from jax.experimental.pallas import tpu as pltpu
from jax.experimental.pallas import tpu_sc as plsc

N = 10000
E = 320000
IN_DIM = 128
D = 129
DP = 144
NP = 10240
NCLS = 70
T = 4
N_LAYERS = 2
STEP = 0.1
VTH = 1.0

NCORES = 2
NSUB = 16
NW = NCORES * NSUB
K = 40
NCHUNK = 256
EPAD = NW * NCHUNK * K
HALF = NCHUNK // 2
NSLOT = 4
ROWS_PER_SUB = NP // NSUB

_mesh = plsc.VectorSubcoreMesh(core_axis_name="c", subcore_axis_name="s")


def _msgpass_body(x_hbm, src_hbm, dst_hbm, out_hbm,
                  src_s, dst_s, r0, r1, r2, r3, acc_sh,
                  g0, g1, g2, g3, s0, s1, s2, s3):
    rows = (r0, r1, r2, r3)
    gs = (g0, g1, g2, g3)
    ss = (s0, s1, s2, s3)
    c = lax.axis_index("c")
    sid = lax.axis_index("s")
    wid = sid * NCORES + c

    def zb(i, _):
        r0[i // (DP // 16), pl.ds((i % (DP // 16)) * 16, 16)] = (
            jnp.zeros((16,), jnp.float32))
        return 0
    lax.fori_loop(0, K * (DP // 16), zb, 0)

    def zcp(j, _):
        pltpu.sync_copy(
            r0, acc_sh.at[pl.ds(sid * ROWS_PER_SUB + j * K, K)])
        return 0
    lax.fori_loop(0, ROWS_PER_SUB // K, zcp, 0)

    plsc.subcore_barrier()

    def half(h, _):
        pltpu.sync_copy(src_hbm.at[wid, pl.ds(h * HALF, HALF)], src_s)
        pltpu.sync_copy(dst_hbm.at[wid, pl.ds(h * HALF, HALF)], dst_s)
        for k in range(NSLOT):
            pltpu.async_copy(x_hbm.at[src_s.at[k]], rows[k], gs[k])

        def rnd(i, _):
            j = NSLOT * i
            for k in range(NSLOT):
                pltpu.make_async_copy(
                    x_hbm.at[src_s.at[j + k]], rows[k], gs[k]).wait()
                pltpu.async_copy(
                    rows[k], acc_sh.at[dst_s.at[j + k]], ss[k], add=True)

            @pl.when(i < HALF // NSLOT - 1)
            def _():
                for k in range(NSLOT):
                    pltpu.make_async_copy(
                        rows[k], acc_sh.at[dst_s.at[j + k]], ss[k]).wait()
                    pltpu.async_copy(
                        x_hbm.at[src_s.at[j + NSLOT + k]], rows[k], gs[k])
            return 0
        lax.fori_loop(0, HALF // NSLOT, rnd, 0)

        for k in range(NSLOT):
            pltpu.make_async_copy(
                rows[k], acc_sh.at[dst_s.at[HALF - NSLOT + k]], ss[k]).wait()
        return 0
    lax.fori_loop(0, NCHUNK // HALF, half, 0)

    plsc.subcore_barrier()
    pltpu.sync_copy(
        acc_sh.at[pl.ds(sid * ROWS_PER_SUB, ROWS_PER_SUB)],
        out_hbm.at[c, pl.ds(sid * ROWS_PER_SUB, ROWS_PER_SUB)])


_msgpass = functools.partial(
    pl.kernel,
    out_type=jax.ShapeDtypeStruct((NCORES, NP, DP), jnp.float32),
    mesh=_mesh,
    scratch_types=[
        pltpu.VMEM((HALF, K), jnp.int32),
        pltpu.VMEM((HALF, K), jnp.int32),
        pltpu.VMEM((K, DP), jnp.float32),
        pltpu.VMEM((K, DP), jnp.float32),
        pltpu.VMEM((K, DP), jnp.float32),
        pltpu.VMEM((K, DP), jnp.float32),
        pltpu.VMEM_SHARED((NP, DP), jnp.float32),
        pltpu.SemaphoreType.DMA,
        pltpu.SemaphoreType.DMA,
        pltpu.SemaphoreType.DMA,
        pltpu.SemaphoreType.DMA,
        pltpu.SemaphoreType.DMA,
        pltpu.SemaphoreType.DMA,
        pltpu.SemaphoreType.DMA,
        pltpu.SemaphoreType.DMA,
    ],
    compiler_params=pltpu.CompilerParams(use_tc_tiling_on_sc=False),
)(_msgpass_body)


def _col_ids(shape):
    return lax.broadcasted_iota(jnp.int32, shape, len(shape) - 1)


def _if_dynamics(agg):
    v = jnp.zeros_like(agg)
    ssum = jnp.zeros_like(agg)
    for _ in range(T):
        v = v + agg
        spk = (v >= VTH).astype(jnp.float32)
        v = v - spk * VTH
        ssum = ssum + spk
    return ssum


def _projx(t):
    ids = _col_ids(t.shape)
    rest = jnp.where((ids >= 1) & (ids < D), t, 0.0)
    rss = jnp.sum(rest * rest, axis=-1, keepdims=True)
    x0 = jnp.sqrt(1.0 + rss)
    return jnp.where(ids == 0, x0, rest)


NB = 8
BR = NP // NB


def _encode_body(f_ref, w_ref, out_ref):
    mm = jnp.dot(f_ref[...], w_ref[...], preferred_element_type=jnp.float32)
    out_ref[...] = mm + (_col_ids(mm.shape) == D).astype(jnp.float32)


_encode = pl.pallas_call(
    _encode_body,
    grid=(NB,),
    in_specs=[
        pl.BlockSpec((BR, IN_DIM), lambda i: (i, 0)),
        pl.BlockSpec((IN_DIM, DP), lambda i: (0, 0)),
    ],
    out_specs=pl.BlockSpec((BR, DP), lambda i: (i, 0)),
    out_shape=jax.ShapeDtypeStruct((NP, DP), jnp.float32),
)


def _stage0_body(p_ref, w_ref, x_ref, z_ref, dinv_ref):
    ssum = p_ref[0] + p_ref[1]
    deg = ssum[:, D:D + 1]
    dinv = 1.0 / jnp.maximum(deg, 1.0)
    agg = ssum * dinv
    xs = _if_dynamics(agg)
    xs = jnp.where(_col_ids(xs.shape) < D, xs, 0.0)
    x_ref[...] = jnp.dot(xs, w_ref[...], preferred_element_type=jnp.float32)
    z_ref[...] = _projx(xs * STEP)
    dinv_ref[...] = dinv


_stage0 = pl.pallas_call(
    _stage0_body,
    grid=(NB,),
    in_specs=[
        pl.BlockSpec((2, BR, DP), lambda i: (0, i, 0)),
        pl.BlockSpec((DP, DP), lambda i: (0, 0)),
    ],
    out_specs=(
        pl.BlockSpec((BR, DP), lambda i: (i, 0)),
        pl.BlockSpec((BR, DP), lambda i: (i, 0)),
        pl.BlockSpec((BR, 1), lambda i: (i, 0)),
    ),
    out_shape=(
        jax.ShapeDtypeStruct((NP, DP), jnp.float32),
        jax.ShapeDtypeStruct((NP, DP), jnp.float32),
        jax.ShapeDtypeStruct((NP, 1), jnp.float32),
    ),
)


def _stage1_body(p_ref, w_ref, z_in_ref, dinv_ref, x_ref, z_ref):
    ssum = p_ref[0] + p_ref[1]
    agg = ssum * dinv_ref[...]
    xs = _if_dynamics(agg)
    xs = jnp.where(_col_ids(xs.shape) < D, xs, 0.0)
    x_ref[...] = jnp.dot(xs, w_ref[...], preferred_element_type=jnp.float32)
    z_ref[...] = _projx(z_in_ref[...] + xs * STEP)


_stage1 = pl.pallas_call(
    _stage1_body,
    grid=(NB,),
    in_specs=[
        pl.BlockSpec((2, BR, DP), lambda i: (0, i, 0)),
        pl.BlockSpec((DP, DP), lambda i: (0, 0)),
        pl.BlockSpec((BR, DP), lambda i: (i, 0)),
        pl.BlockSpec((BR, 1), lambda i: (i, 0)),
    ],
    out_specs=(
        pl.BlockSpec((BR, DP), lambda i: (i, 0)),
        pl.BlockSpec((BR, DP), lambda i: (i, 0)),
    ),
    out_shape=(
        jax.ShapeDtypeStruct((NP, DP), jnp.float32),
        jax.ShapeDtypeStruct((NP, DP), jnp.float32),
    ),
)


def _final_body(p_ref, z_in_ref, dinv_ref, pts_ref, out_ref):
    ssum = p_ref[0] + p_ref[1]
    agg = ssum * dinv_ref[...]
    xs = _if_dynamics(agg)
    xs = jnp.where(_col_ids(xs.shape) < D, xs, 0.0)
    z = _projx(z_in_ref[...] + xs * STEP)
    q = _projx(pts_ref[...])
    ids = _col_ids(z.shape)
    zneg = jnp.where(ids == 0, -z, z)
    inner = lax.dot_general(zneg, q, (((1,), (1,)), ((), ())),
                            preferred_element_type=jnp.float32)
    c = jnp.maximum(-inner, 1.0 + 1e-5)
    out_ref[...] = -jnp.log(c + jnp.sqrt((c - 1.0) * (c + 1.0)))


_final = pl.pallas_call(
    _final_body,
    grid=(NB,),
    in_specs=[
        pl.BlockSpec((2, BR, DP), lambda i: (0, i, 0)),
        pl.BlockSpec((BR, DP), lambda i: (i, 0)),
        pl.BlockSpec((BR, 1), lambda i: (i, 0)),
        pl.BlockSpec((128, DP), lambda i: (0, 0)),
    ],
    out_specs=pl.BlockSpec((BR, 128), lambda i: (i, 0)),
    out_shape=jax.ShapeDtypeStruct((NP, 128), jnp.float32),
)


def kernel(features, edge_index, W_enc, W_layers, points):
    pad = N + (jnp.arange(EPAD - E, dtype=jnp.int32) % (NP - N))
    src = jnp.concatenate([edge_index[0], pad]).reshape(NW, NCHUNK, K)
    dst = jnp.concatenate([edge_index[1], pad]).reshape(NW, NCHUNK, K)
    f_p = jnp.zeros((NP, IN_DIM), jnp.float32).at[:N].set(features)
    we_p = jnp.zeros((IN_DIM, DP), jnp.float32).at[:, :D].set(W_enc)
    wl_p = jnp.zeros((N_LAYERS, DP, DP), jnp.float32).at[:, :D, :D].set(W_layers)
    pts_p = jnp.zeros((128, DP), jnp.float32).at[:NCLS, :D].set(points)

    x0 = _encode(f_p, we_p)
    p0 = _msgpass(x0, src, dst)
    x1, z0, dinv = _stage0(p0, wl_p[0])
    p1 = _msgpass(x1, src, dst)
    x2, z1 = _stage1(p1, wl_p[1], z0, dinv)
    p2 = _msgpass(x2, src, dst)
    return _final(p2, z1, dinv, pts_p)[:N, :NCLS]


# speedup vs baseline: 10.4549x; 1.0656x over previous
import functools

import jax
import jax.numpy as jnp
from jax import lax
from jax.experimental import pallas as pl
from jax.experimental.pallas import tpu as pltpu
from jax.experimental.pallas import tpu_sc as plsc

N = 10000
E = 320000
IN_DIM = 128
D = 129
DP = 144
DS = 160
NP = 10240
NCLS = 70
T = 4
N_LAYERS = 2
STEP = 0.1
VTH = 1.0

NCORES = 2
NSUB = 16
NW = NCORES * NSUB
K = 40
NCHUNK = 256
EPAD = NW * NCHUNK * K
HALF = NCHUNK // 2
NSLOT = 4
ROWS_PER_SUB = NP // NSUB

_mesh = plsc.VectorSubcoreMesh(core_axis_name="c", subcore_axis_name="s")


def _make_msgpass(dtype, width, lanes):
    zsteps = width // lanes

    def body(x_hbm, src_hbm, dst_hbm, out_hbm,
             src_s, dst_s, r0, r1, r2, r3, acc_sh,
             g0, g1, g2, g3, s0, s1, s2, s3):
        rows = (r0, r1, r2, r3)
        gs = (g0, g1, g2, g3)
        ss = (s0, s1, s2, s3)
        c = lax.axis_index("c")
        sid = lax.axis_index("s")
        wid = sid * NCORES + c

        def zb(i, _):
            r0[i // zsteps, pl.ds((i % zsteps) * lanes, lanes)] = (
                jnp.zeros((lanes,), dtype))
            return 0
        lax.fori_loop(0, K * zsteps, zb, 0)

        def zcp(j, _):
            pltpu.sync_copy(
                r0, acc_sh.at[pl.ds(sid * ROWS_PER_SUB + j * K, K)])
            return 0
        lax.fori_loop(0, ROWS_PER_SUB // K, zcp, 0)

        plsc.subcore_barrier()

        def half(h, _):
            pltpu.sync_copy(src_hbm.at[wid, pl.ds(h * HALF, HALF)], src_s)
            pltpu.sync_copy(dst_hbm.at[wid, pl.ds(h * HALF, HALF)], dst_s)
            for k in range(NSLOT):
                pltpu.async_copy(x_hbm.at[src_s.at[k]], rows[k], gs[k])

            def rnd(i, _):
                j = NSLOT * i
                for k in range(NSLOT):
                    pltpu.make_async_copy(
                        x_hbm.at[src_s.at[j + k]], rows[k], gs[k]).wait()
                    pltpu.async_copy(
                        rows[k], acc_sh.at[dst_s.at[j + k]], ss[k], add=True)

                @pl.when(i < HALF // NSLOT - 1)
                def _():
                    for k in range(NSLOT):
                        pltpu.make_async_copy(
                            rows[k], acc_sh.at[dst_s.at[j + k]],
                            ss[k]).wait()
                        pltpu.async_copy(
                            x_hbm.at[src_s.at[j + NSLOT + k]],
                            rows[k], gs[k])
                return 0
            lax.fori_loop(0, HALF // NSLOT, rnd, 0)

            for k in range(NSLOT):
                pltpu.make_async_copy(
                    rows[k], acc_sh.at[dst_s.at[HALF - NSLOT + k]],
                    ss[k]).wait()
            return 0
        lax.fori_loop(0, NCHUNK // HALF, half, 0)

        plsc.subcore_barrier()
        pltpu.sync_copy(
            acc_sh.at[pl.ds(sid * ROWS_PER_SUB, ROWS_PER_SUB)],
            out_hbm.at[c, pl.ds(sid * ROWS_PER_SUB, ROWS_PER_SUB)])

    return functools.partial(
        pl.kernel,
        out_type=jax.ShapeDtypeStruct((NCORES, NP, width), dtype),
        mesh=_mesh,
        scratch_types=[
            pltpu.VMEM((HALF, K), jnp.int32),
            pltpu.VMEM((HALF, K), jnp.int32),
            pltpu.VMEM((K, width), dtype),
            pltpu.VMEM((K, width), dtype),
            pltpu.VMEM((K, width), dtype),
            pltpu.VMEM((K, width), dtype),
            pltpu.VMEM_SHARED((NP, width), dtype),
            pltpu.SemaphoreType.DMA,
            pltpu.SemaphoreType.DMA,
            pltpu.SemaphoreType.DMA,
            pltpu.SemaphoreType.DMA,
            pltpu.SemaphoreType.DMA,
            pltpu.SemaphoreType.DMA,
            pltpu.SemaphoreType.DMA,
            pltpu.SemaphoreType.DMA,
        ],
        compiler_params=pltpu.CompilerParams(use_tc_tiling_on_sc=False),
    )(body)


_msgpass_f32 = _make_msgpass(jnp.float32, DP, 16)
_msgpass_s16 = _make_msgpass(jnp.int16, DS, 32)


def _col_ids(shape):
    return lax.broadcasted_iota(jnp.int32, shape, len(shape) - 1)


def _if_dynamics(agg):
    v = jnp.zeros_like(agg)
    ssum = jnp.zeros_like(agg)
    for _ in range(T):
        v = v + agg
        spk = (v >= VTH).astype(jnp.float32)
        v = v - spk * VTH
        ssum = ssum + spk
    return ssum


def _projx(t):
    ids = _col_ids(t.shape)
    rest = jnp.where((ids >= 1) & (ids < D), t, 0.0)
    rss = jnp.sum(rest * rest, axis=-1, keepdims=True)
    x0 = jnp.sqrt(1.0 + rss)
    return jnp.where(ids == 0, x0, rest)


def _to_s16_padded(xs):
    if xs.shape[1] == DS:
        return xs.astype(jnp.int16)
    return jnp.concatenate(
        [xs.astype(jnp.int16),
         jnp.zeros((xs.shape[0], DS - xs.shape[1]), jnp.int16)], axis=1)


NB = 8
BR = NP // NB


def _encode_body(f_ref, w_ref, out_ref):
    mm = jnp.dot(f_ref[...], w_ref[...], preferred_element_type=jnp.float32)
    out_ref[...] = mm + (_col_ids(mm.shape) == D).astype(jnp.float32)


_encode = pl.pallas_call(
    _encode_body,
    grid=(NB,),
    in_specs=[
        pl.BlockSpec((BR, IN_DIM), lambda i: (i, 0)),
        pl.BlockSpec((IN_DIM, DP), lambda i: (0, 0)),
    ],
    out_specs=pl.BlockSpec((BR, DP), lambda i: (i, 0)),
    out_shape=jax.ShapeDtypeStruct((NP, DP), jnp.float32),
)


def _stage0_body(p_ref, x_ref, z_ref, dinv_ref):
    ssum = p_ref[0] + p_ref[1]
    deg = ssum[:, D:D + 1]
    dinv = 1.0 / jnp.maximum(deg, 1.0)
    agg = ssum * dinv
    xs = _if_dynamics(agg)
    xs = jnp.where(_col_ids(xs.shape) < D, xs, 0.0)
    x_ref[...] = _to_s16_padded(xs)
    z_ref[...] = _projx(xs * STEP)
    dinv_ref[...] = dinv


_stage0 = pl.pallas_call(
    _stage0_body,
    grid=(NB,),
    in_specs=[
        pl.BlockSpec((2, BR, DP), lambda i: (0, i, 0)),
    ],
    out_specs=(
        pl.BlockSpec((BR, DS), lambda i: (i, 0)),
        pl.BlockSpec((BR, DP), lambda i: (i, 0)),
        pl.BlockSpec((BR, 1), lambda i: (i, 0)),
    ),
    out_shape=(
        jax.ShapeDtypeStruct((NP, DS), jnp.int16),
        jax.ShapeDtypeStruct((NP, DP), jnp.float32),
        jax.ShapeDtypeStruct((NP, 1), jnp.float32),
    ),
)


def _layer_agg(p_ref, w_ref, dinv_ref):
    m = (p_ref[0] + p_ref[1]).astype(jnp.float32)
    mm = jnp.dot(m, w_ref[...], preferred_element_type=jnp.float32)
    agg = mm * dinv_ref[...]
    xs = _if_dynamics(agg)
    return jnp.where(_col_ids(xs.shape) < D, xs, 0.0)


def _stage1_body(p_ref, w_ref, z_in_ref, dinv_ref, x_ref, z_ref):
    xs = _layer_agg(p_ref, w_ref, dinv_ref)
    x_ref[...] = _to_s16_padded(xs)
    z_ref[...] = _projx(z_in_ref[...] + xs[:, :DP] * STEP)


_stage1 = pl.pallas_call(
    _stage1_body,
    grid=(NB,),
    in_specs=[
        pl.BlockSpec((2, BR, DS), lambda i: (0, i, 0)),
        pl.BlockSpec((DS, DS), lambda i: (0, 0)),
        pl.BlockSpec((BR, DP), lambda i: (i, 0)),
        pl.BlockSpec((BR, 1), lambda i: (i, 0)),
    ],
    out_specs=(
        pl.BlockSpec((BR, DS), lambda i: (i, 0)),
        pl.BlockSpec((BR, DP), lambda i: (i, 0)),
    ),
    out_shape=(
        jax.ShapeDtypeStruct((NP, DS), jnp.int16),
        jax.ShapeDtypeStruct((NP, DP), jnp.float32),
    ),
)


def _final_body(p_ref, w_ref, z_in_ref, dinv_ref, pts_ref, out_ref):
    xs = _layer_agg(p_ref, w_ref, dinv_ref)
    z = _projx(z_in_ref[...] + xs[:, :DP] * STEP)
    q = _projx(pts_ref[...])
    ids = _col_ids(z.shape)
    zneg = jnp.where(ids == 0, -z, z)
    inner = lax.dot_general(zneg, q, (((1,), (1,)), ((), ())),
                            preferred_element_type=jnp.float32)
    c = jnp.maximum(-inner, 1.0 + 1e-5)
    out_ref[...] = -jnp.log(c + jnp.sqrt((c - 1.0) * (c + 1.0)))


_final = pl.pallas_call(
    _final_body,
    grid=(NB,),
    in_specs=[
        pl.BlockSpec((2, BR, DS), lambda i: (0, i, 0)),
        pl.BlockSpec((DS, DS), lambda i: (0, 0)),
        pl.BlockSpec((BR, DP), lambda i: (i, 0)),
        pl.BlockSpec((BR, 1), lambda i: (i, 0)),
        pl.BlockSpec((128, DP), lambda i: (0, 0)),
    ],
    out_specs=pl.BlockSpec((BR, 128), lambda i: (i, 0)),
    out_shape=jax.ShapeDtypeStruct((NP, 128), jnp.float32),
)


def kernel(features, edge_index, W_enc, W_layers, points):
    pad = N + (jnp.arange(EPAD - E, dtype=jnp.int32) % (NP - N))
    src = jnp.concatenate([edge_index[0], pad]).reshape(NW, NCHUNK, K)
    dst = jnp.concatenate([edge_index[1], pad]).reshape(NW, NCHUNK, K)
    f_p = jnp.zeros((NP, IN_DIM), jnp.float32).at[:N].set(features)
    we_p = jnp.zeros((IN_DIM, DP), jnp.float32).at[:, :D].set(W_enc)
    wl_p = jnp.zeros((N_LAYERS, DS, DS), jnp.float32).at[:, :D, :D].set(W_layers)
    pts_p = jnp.zeros((128, DP), jnp.float32).at[:NCLS, :D].set(points)

    x0 = _encode(f_p, we_p)
    p0 = _msgpass_f32(x0, src, dst)
    x1, z0, dinv = _stage0(p0)
    p1 = _msgpass_s16(x1, src, dst)
    x2, z1 = _stage1(p1, wl_p[0], z0, dinv)
    p2 = _msgpass_s16(x2, src, dst)
    return _final(p2, wl_p[1], z1, dinv, pts_p)[:N, :NCLS]


# speedup vs baseline: 10.6730x; 1.0209x over previous
import functools

import jax
import jax.numpy as jnp
from jax import lax
from jax.experimental import pallas as pl
from jax.experimental.pallas import tpu as pltpu
from jax.experimental.pallas import tpu_sc as plsc

N = 10000
E = 320000
IN_DIM = 128
D = 129
DP = 144
DS = 160
NP = 10240
NCLS = 70
T = 4
N_LAYERS = 2
STEP = 0.1
VTH = 1.0

NCORES = 2
NSUB = 16
NW = NCORES * NSUB
EPW = E // NW
K = 40
NCHUNK = EPW // K
NSLOT = 5
BLK = 50
NBLK = NCHUNK // BLK
ROWS_PER_SUB = NP // NSUB

_mesh = plsc.VectorSubcoreMesh(core_axis_name="c", subcore_axis_name="s")


def _make_msgpass(dtype, width, lanes):
    zsteps = width // lanes

    def body(x_hbm, src_hbm, dst_hbm, out_hbm,
             src_s, dst_s, r0, r1, r2, r3, r4, acc_sh,
             g0, g1, g2, g3, g4, s0, s1, s2, s3, s4):
        rows = (r0, r1, r2, r3, r4)
        gs = (g0, g1, g2, g3, g4)
        ss = (s0, s1, s2, s3, s4)
        c = lax.axis_index("c")
        sid = lax.axis_index("s")
        wid = sid * NCORES + c

        def zb(i, _):
            r0[i // zsteps, pl.ds((i % zsteps) * lanes, lanes)] = (
                jnp.zeros((lanes,), dtype))
            return 0
        lax.fori_loop(0, K * zsteps, zb, 0)

        def zcp(j, _):
            pltpu.sync_copy(
                r0, acc_sh.at[pl.ds(sid * ROWS_PER_SUB + j * K, K)])
            return 0
        lax.fori_loop(0, ROWS_PER_SUB // K, zcp, 0)

        plsc.subcore_barrier()

        def blk(b, _):
            base = wid * EPW + b * BLK * K
            pltpu.sync_copy(src_hbm.at[pl.ds(base, BLK * K)], src_s)
            pltpu.sync_copy(dst_hbm.at[pl.ds(base, BLK * K)], dst_s)
            for k in range(NSLOT):
                pltpu.async_copy(
                    x_hbm.at[src_s.at[pl.ds(k * K, K)]], rows[k], gs[k])

            def rnd(i, _):
                j = NSLOT * i
                for k in range(NSLOT):
                    pltpu.make_async_copy(
                        x_hbm.at[src_s.at[pl.ds((j + k) * K, K)]],
                        rows[k], gs[k]).wait()
                    pltpu.async_copy(
                        rows[k], acc_sh.at[dst_s.at[pl.ds((j + k) * K, K)]],
                        ss[k], add=True)

                @pl.when(i < BLK // NSLOT - 1)
                def _():
                    for k in range(NSLOT):
                        pltpu.make_async_copy(
                            rows[k],
                            acc_sh.at[dst_s.at[pl.ds((j + k) * K, K)]],
                            ss[k]).wait()
                        pltpu.async_copy(
                            x_hbm.at[src_s.at[pl.ds((j + NSLOT + k) * K, K)]],
                            rows[k], gs[k])
                return 0
            lax.fori_loop(0, BLK // NSLOT, rnd, 0)

            for k in range(NSLOT):
                pltpu.make_async_copy(
                    rows[k],
                    acc_sh.at[dst_s.at[pl.ds((BLK - NSLOT + k) * K, K)]],
                    ss[k]).wait()
            return 0
        lax.fori_loop(0, NBLK, blk, 0)

        plsc.subcore_barrier()
        pltpu.sync_copy(
            acc_sh.at[pl.ds(sid * ROWS_PER_SUB, ROWS_PER_SUB)],
            out_hbm.at[c, pl.ds(sid * ROWS_PER_SUB, ROWS_PER_SUB)])

    return functools.partial(
        pl.kernel,
        out_type=jax.ShapeDtypeStruct((NCORES, NP, width), dtype),
        mesh=_mesh,
        scratch_types=[
            pltpu.VMEM((BLK * K,), jnp.int32),
            pltpu.VMEM((BLK * K,), jnp.int32),
            pltpu.VMEM((K, width), dtype),
            pltpu.VMEM((K, width), dtype),
            pltpu.VMEM((K, width), dtype),
            pltpu.VMEM((K, width), dtype),
            pltpu.VMEM((K, width), dtype),
            pltpu.VMEM_SHARED((NP, width), dtype),
            pltpu.SemaphoreType.DMA,
            pltpu.SemaphoreType.DMA,
            pltpu.SemaphoreType.DMA,
            pltpu.SemaphoreType.DMA,
            pltpu.SemaphoreType.DMA,
            pltpu.SemaphoreType.DMA,
            pltpu.SemaphoreType.DMA,
            pltpu.SemaphoreType.DMA,
            pltpu.SemaphoreType.DMA,
            pltpu.SemaphoreType.DMA,
        ],
        compiler_params=pltpu.CompilerParams(use_tc_tiling_on_sc=False),
    )(body)


_msgpass_f32 = _make_msgpass(jnp.float32, DP, 16)
_msgpass_s16 = _make_msgpass(jnp.int16, DS, 32)


def _col_ids(shape):
    return lax.broadcasted_iota(jnp.int32, shape, len(shape) - 1)


def _if_dynamics(agg):
    v = jnp.zeros_like(agg)
    ssum = jnp.zeros_like(agg)
    for _ in range(T):
        v = v + agg
        spk = (v >= VTH).astype(jnp.float32)
        v = v - spk * VTH
        ssum = ssum + spk
    return ssum


def _projx(t):
    ids = _col_ids(t.shape)
    rest = jnp.where((ids >= 1) & (ids < D), t, 0.0)
    rss = jnp.sum(rest * rest, axis=-1, keepdims=True)
    x0 = jnp.sqrt(1.0 + rss)
    return jnp.where(ids == 0, x0, rest)


def _to_s16_padded(xs):
    if xs.shape[1] == DS:
        return xs.astype(jnp.int16)
    return jnp.concatenate(
        [xs.astype(jnp.int16),
         jnp.zeros((xs.shape[0], DS - xs.shape[1]), jnp.int16)], axis=1)


NB = 8
BR = NP // NB


def _encode_body(f_ref, w_ref, out_ref):
    mm = jnp.dot(f_ref[...], w_ref[...], preferred_element_type=jnp.float32)
    out_ref[...] = mm + (_col_ids(mm.shape) == D).astype(jnp.float32)


_encode = pl.pallas_call(
    _encode_body,
    grid=(NB,),
    in_specs=[
        pl.BlockSpec((BR, IN_DIM), lambda i: (i, 0)),
        pl.BlockSpec((IN_DIM, DP), lambda i: (0, 0)),
    ],
    out_specs=pl.BlockSpec((BR, DP), lambda i: (i, 0)),
    out_shape=jax.ShapeDtypeStruct((NP, DP), jnp.float32),
)


def _stage0_body(p_ref, x_ref, z_ref, dinv_ref):
    ssum = p_ref[0] + p_ref[1]
    deg = ssum[:, D:D + 1]
    dinv = 1.0 / jnp.maximum(deg, 1.0)
    agg = ssum * dinv
    xs = _if_dynamics(agg)
    xs = jnp.where(_col_ids(xs.shape) < D, xs, 0.0)
    x_ref[...] = _to_s16_padded(xs)
    z_ref[...] = _projx(xs * STEP)
    dinv_ref[...] = dinv


_stage0 = pl.pallas_call(
    _stage0_body,
    grid=(NB,),
    in_specs=[
        pl.BlockSpec((2, BR, DP), lambda i: (0, i, 0)),
    ],
    out_specs=(
        pl.BlockSpec((BR, DS), lambda i: (i, 0)),
        pl.BlockSpec((BR, DP), lambda i: (i, 0)),
        pl.BlockSpec((BR, 1), lambda i: (i, 0)),
    ),
    out_shape=(
        jax.ShapeDtypeStruct((NP, DS), jnp.int16),
        jax.ShapeDtypeStruct((NP, DP), jnp.float32),
        jax.ShapeDtypeStruct((NP, 1), jnp.float32),
    ),
)


def _layer_agg(p_ref, w_ref, dinv_ref):
    m = (p_ref[0] + p_ref[1]).astype(jnp.float32)
    mm = jnp.dot(m, w_ref[...], preferred_element_type=jnp.float32)
    agg = mm * dinv_ref[...]
    xs = _if_dynamics(agg)
    return jnp.where(_col_ids(xs.shape) < D, xs, 0.0)


def _stage1_body(p_ref, w_ref, z_in_ref, dinv_ref, x_ref, z_ref):
    xs = _layer_agg(p_ref, w_ref, dinv_ref)
    x_ref[...] = _to_s16_padded(xs)
    z_ref[...] = _projx(z_in_ref[...] + xs[:, :DP] * STEP)


_stage1 = pl.pallas_call(
    _stage1_body,
    grid=(NB,),
    in_specs=[
        pl.BlockSpec((2, BR, DS), lambda i: (0, i, 0)),
        pl.BlockSpec((DS, DS), lambda i: (0, 0)),
        pl.BlockSpec((BR, DP), lambda i: (i, 0)),
        pl.BlockSpec((BR, 1), lambda i: (i, 0)),
    ],
    out_specs=(
        pl.BlockSpec((BR, DS), lambda i: (i, 0)),
        pl.BlockSpec((BR, DP), lambda i: (i, 0)),
    ),
    out_shape=(
        jax.ShapeDtypeStruct((NP, DS), jnp.int16),
        jax.ShapeDtypeStruct((NP, DP), jnp.float32),
    ),
)


def _final_body(p_ref, w_ref, z_in_ref, dinv_ref, pts_ref, out_ref):
    xs = _layer_agg(p_ref, w_ref, dinv_ref)
    z = _projx(z_in_ref[...] + xs[:, :DP] * STEP)
    q = _projx(pts_ref[...])
    ids = _col_ids(z.shape)
    zneg = jnp.where(ids == 0, -z, z)
    inner = lax.dot_general(zneg, q, (((1,), (1,)), ((), ())),
                            preferred_element_type=jnp.float32)
    c = jnp.maximum(-inner, 1.0 + 1e-5)
    out_ref[...] = -jnp.log(c + jnp.sqrt((c - 1.0) * (c + 1.0)))


_final = pl.pallas_call(
    _final_body,
    grid=(NB,),
    in_specs=[
        pl.BlockSpec((2, BR, DS), lambda i: (0, i, 0)),
        pl.BlockSpec((DS, DS), lambda i: (0, 0)),
        pl.BlockSpec((BR, DP), lambda i: (i, 0)),
        pl.BlockSpec((BR, 1), lambda i: (i, 0)),
        pl.BlockSpec((128, DP), lambda i: (0, 0)),
    ],
    out_specs=pl.BlockSpec((BR, 128), lambda i: (i, 0)),
    out_shape=jax.ShapeDtypeStruct((NP, 128), jnp.float32),
)


def kernel(features, edge_index, W_enc, W_layers, points):
    src = edge_index[0]
    dst = edge_index[1]
    f_p = jnp.zeros((NP, IN_DIM), jnp.float32).at[:N].set(features)
    we_p = jnp.zeros((IN_DIM, DP), jnp.float32).at[:, :D].set(W_enc)
    wl_p = jnp.zeros((N_LAYERS, DS, DS), jnp.float32).at[:, :D, :D].set(W_layers)
    pts_p = jnp.zeros((128, DP), jnp.float32).at[:NCLS, :D].set(points)

    x0 = _encode(f_p, we_p)
    p0 = _msgpass_f32(x0, src, dst)
    x1, z0, dinv = _stage0(p0)
    p1 = _msgpass_s16(x1, src, dst)
    x2, z1 = _stage1(p1, wl_p[0], z0, dinv)
    p2 = _msgpass_s16(x2, src, dst)
    return _final(p2, wl_p[1], z1, dinv, pts_p)[:N, :NCLS]


# speedup vs baseline: 11.0282x; 1.0333x over previous
import functools

import jax
import jax.numpy as jnp
from jax import lax
from jax.experimental import pallas as pl
from jax.experimental.pallas import tpu as pltpu
from jax.experimental.pallas import tpu_sc as plsc

N = 10000
E = 320000
IN_DIM = 128
D = 129
DP = 144
DS = 160
NP = 10240
NCLS = 70
T = 4
N_LAYERS = 2
STEP = 0.1
VTH = 1.0

NCORES = 2
NSUB = 16
NW = NCORES * NSUB
EPW = E // NW
NSLOT = 5
ROWS_PER_SUB = NP // NSUB

_mesh = plsc.VectorSubcoreMesh(core_axis_name="c", subcore_axis_name="s")


def _make_msgpass(dtype, width, lanes, K, BLK):
    NBLK = EPW // (BLK * K)
    zsteps = width // lanes

    def body(edges_hbm, x_hbm, out_hbm,
             src_s, dst_s, r0, r1, r2, r3, r4, acc_sh,
             g0, g1, g2, g3, g4, s0, s1, s2, s3, s4):
        rows = (r0, r1, r2, r3, r4)
        gs = (g0, g1, g2, g3, g4)
        ss = (s0, s1, s2, s3, s4)
        c = lax.axis_index("c")
        sid = lax.axis_index("s")
        wid = sid * NCORES + c

        def zb(i, _):
            r0[i // zsteps, pl.ds((i % zsteps) * lanes, lanes)] = (
                jnp.zeros((lanes,), dtype))
            return 0
        lax.fori_loop(0, K * zsteps, zb, 0)

        def zcp(j, _):
            pltpu.sync_copy(
                r0, acc_sh.at[pl.ds(sid * ROWS_PER_SUB + j * K, K)])
            return 0
        lax.fori_loop(0, ROWS_PER_SUB // K, zcp, 0)

        plsc.subcore_barrier()

        def blk(b, _):
            base = wid * EPW + b * BLK * K
            pltpu.sync_copy(edges_hbm.at[0, pl.ds(base, BLK * K)], src_s)
            pltpu.sync_copy(edges_hbm.at[1, pl.ds(base, BLK * K)], dst_s)
            for k in range(NSLOT):
                pltpu.async_copy(
                    x_hbm.at[src_s.at[pl.ds(k * K, K)]], rows[k], gs[k])

            def rnd(i, _):
                j = NSLOT * i
                for k in range(NSLOT):
                    pltpu.make_async_copy(
                        x_hbm.at[src_s.at[pl.ds((j + k) * K, K)]],
                        rows[k], gs[k]).wait()
                    pltpu.async_copy(
                        rows[k], acc_sh.at[dst_s.at[pl.ds((j + k) * K, K)]],
                        ss[k], add=True)

                @pl.when(i < BLK // NSLOT - 1)
                def _():
                    for k in range(NSLOT):
                        pltpu.make_async_copy(
                            rows[k],
                            acc_sh.at[dst_s.at[pl.ds((j + k) * K, K)]],
                            ss[k]).wait()
                        pltpu.async_copy(
                            x_hbm.at[src_s.at[pl.ds((j + NSLOT + k) * K, K)]],
                            rows[k], gs[k])
                return 0
            lax.fori_loop(0, BLK // NSLOT, rnd, 0)

            for k in range(NSLOT):
                pltpu.make_async_copy(
                    rows[k],
                    acc_sh.at[dst_s.at[pl.ds((BLK - NSLOT + k) * K, K)]],
                    ss[k]).wait()
            return 0
        lax.fori_loop(0, NBLK, blk, 0)

        plsc.subcore_barrier()
        pltpu.sync_copy(
            acc_sh.at[pl.ds(sid * ROWS_PER_SUB, ROWS_PER_SUB)],
            out_hbm.at[c, pl.ds(sid * ROWS_PER_SUB, ROWS_PER_SUB)])

    return functools.partial(
        pl.kernel,
        out_type=jax.ShapeDtypeStruct((NCORES, NP, width), dtype),
        mesh=_mesh,
        scratch_types=[
            pltpu.VMEM((BLK * K,), jnp.int32),
            pltpu.VMEM((BLK * K,), jnp.int32),
            pltpu.VMEM((K, width), dtype),
            pltpu.VMEM((K, width), dtype),
            pltpu.VMEM((K, width), dtype),
            pltpu.VMEM((K, width), dtype),
            pltpu.VMEM((K, width), dtype),
            pltpu.VMEM_SHARED((NP, width), dtype),
            pltpu.SemaphoreType.DMA,
            pltpu.SemaphoreType.DMA,
            pltpu.SemaphoreType.DMA,
            pltpu.SemaphoreType.DMA,
            pltpu.SemaphoreType.DMA,
            pltpu.SemaphoreType.DMA,
            pltpu.SemaphoreType.DMA,
            pltpu.SemaphoreType.DMA,
            pltpu.SemaphoreType.DMA,
            pltpu.SemaphoreType.DMA,
        ],
        compiler_params=pltpu.CompilerParams(use_tc_tiling_on_sc=False),
    )(body)


_msgpass_f32 = _make_msgpass(jnp.float32, DP, 16, K=40, BLK=50)
_msgpass_s16 = _make_msgpass(jnp.int16, DS, 32, K=80, BLK=25)


def _col_ids(shape):
    return lax.broadcasted_iota(jnp.int32, shape, len(shape) - 1)


def _if_dynamics(agg):
    v = jnp.zeros_like(agg)
    ssum = jnp.zeros_like(agg)
    for _ in range(T):
        v = v + agg
        spk = (v >= VTH).astype(jnp.float32)
        v = v - spk * VTH
        ssum = ssum + spk
    return ssum


def _projx(t):
    ids = _col_ids(t.shape)
    rest = jnp.where((ids >= 1) & (ids < D), t, 0.0)
    rss = jnp.sum(rest * rest, axis=-1, keepdims=True)
    x0 = jnp.sqrt(1.0 + rss)
    return jnp.where(ids == 0, x0, rest)


def _to_s16_padded(xs):
    if xs.shape[1] == DS:
        return xs.astype(jnp.int16)
    return jnp.concatenate(
        [xs.astype(jnp.int16),
         jnp.zeros((xs.shape[0], DS - xs.shape[1]), jnp.int16)], axis=1)


NB = 8
BR = NP // NB


def _encode_body(f_ref, w_ref, out_ref):
    mm = jnp.dot(f_ref[...], w_ref[...], preferred_element_type=jnp.float32)
    out_ref[...] = mm + (_col_ids(mm.shape) == D).astype(jnp.float32)


_encode = pl.pallas_call(
    _encode_body,
    grid=(NB,),
    in_specs=[
        pl.BlockSpec((BR, IN_DIM), lambda i: (i, 0)),
        pl.BlockSpec((IN_DIM, DP), lambda i: (0, 0)),
    ],
    out_specs=pl.BlockSpec((BR, DP), lambda i: (i, 0)),
    out_shape=jax.ShapeDtypeStruct((NP, DP), jnp.float32),
)


def _stage0_body(p_ref, x_ref, z_ref, dinv_ref):
    ssum = p_ref[0] + p_ref[1]
    deg = ssum[:, D:D + 1]
    dinv = 1.0 / jnp.maximum(deg, 1.0)
    agg = ssum * dinv
    xs = _if_dynamics(agg)
    xs = jnp.where(_col_ids(xs.shape) < D, xs, 0.0)
    x_ref[...] = _to_s16_padded(xs)
    z_ref[...] = _projx(xs * STEP)
    dinv_ref[...] = dinv


_stage0 = pl.pallas_call(
    _stage0_body,
    grid=(NB,),
    in_specs=[
        pl.BlockSpec((2, BR, DP), lambda i: (0, i, 0)),
    ],
    out_specs=(
        pl.BlockSpec((BR, DS), lambda i: (i, 0)),
        pl.BlockSpec((BR, DP), lambda i: (i, 0)),
        pl.BlockSpec((BR, 1), lambda i: (i, 0)),
    ),
    out_shape=(
        jax.ShapeDtypeStruct((NP, DS), jnp.int16),
        jax.ShapeDtypeStruct((NP, DP), jnp.float32),
        jax.ShapeDtypeStruct((NP, 1), jnp.float32),
    ),
)


def _layer_agg(p_ref, w_ref, dinv_ref):
    m = (p_ref[0] + p_ref[1]).astype(jnp.float32)
    mm = jnp.dot(m, w_ref[...], preferred_element_type=jnp.float32)
    agg = mm * dinv_ref[...]
    xs = _if_dynamics(agg)
    return jnp.where(_col_ids(xs.shape) < D, xs, 0.0)


def _stage1_body(p_ref, w_ref, z_in_ref, dinv_ref, x_ref, z_ref):
    xs = _layer_agg(p_ref, w_ref, dinv_ref)
    x_ref[...] = _to_s16_padded(xs)
    z_ref[...] = _projx(z_in_ref[...] + xs[:, :DP] * STEP)


_stage1 = pl.pallas_call(
    _stage1_body,
    grid=(NB,),
    in_specs=[
        pl.BlockSpec((2, BR, DS), lambda i: (0, i, 0)),
        pl.BlockSpec((DS, DS), lambda i: (0, 0)),
        pl.BlockSpec((BR, DP), lambda i: (i, 0)),
        pl.BlockSpec((BR, 1), lambda i: (i, 0)),
    ],
    out_specs=(
        pl.BlockSpec((BR, DS), lambda i: (i, 0)),
        pl.BlockSpec((BR, DP), lambda i: (i, 0)),
    ),
    out_shape=(
        jax.ShapeDtypeStruct((NP, DS), jnp.int16),
        jax.ShapeDtypeStruct((NP, DP), jnp.float32),
    ),
)


def _final_body(p_ref, w_ref, z_in_ref, dinv_ref, pts_ref, out_ref):
    xs = _layer_agg(p_ref, w_ref, dinv_ref)
    z = _projx(z_in_ref[...] + xs[:, :DP] * STEP)
    q = _projx(pts_ref[...])
    ids = _col_ids(z.shape)
    zneg = jnp.where(ids == 0, -z, z)
    inner = lax.dot_general(zneg, q, (((1,), (1,)), ((), ())),
                            preferred_element_type=jnp.float32)
    c = jnp.maximum(-inner, 1.0 + 1e-5)
    out_ref[...] = -jnp.log(c + jnp.sqrt((c - 1.0) * (c + 1.0)))


_final = pl.pallas_call(
    _final_body,
    grid=(NB,),
    in_specs=[
        pl.BlockSpec((2, BR, DS), lambda i: (0, i, 0)),
        pl.BlockSpec((DS, DS), lambda i: (0, 0)),
        pl.BlockSpec((BR, DP), lambda i: (i, 0)),
        pl.BlockSpec((BR, 1), lambda i: (i, 0)),
        pl.BlockSpec((128, DP), lambda i: (0, 0)),
    ],
    out_specs=pl.BlockSpec((BR, 128), lambda i: (i, 0)),
    out_shape=jax.ShapeDtypeStruct((NP, 128), jnp.float32),
)


def kernel(features, edge_index, W_enc, W_layers, points):
    f_p = jnp.zeros((NP, IN_DIM), jnp.float32).at[:N].set(features)
    we_p = jnp.zeros((IN_DIM, DP), jnp.float32).at[:, :D].set(W_enc)
    wl_p = jnp.zeros((N_LAYERS, DS, DS), jnp.float32).at[:, :D, :D].set(W_layers)
    pts_p = jnp.zeros((128, DP), jnp.float32).at[:NCLS, :D].set(points)

    x0 = _encode(f_p, we_p)
    p0 = _msgpass_f32(edge_index, x0)
    x1, z0, dinv = _stage0(p0)
    p1 = _msgpass_s16(edge_index, x1)
    x2, z1 = _stage1(p1, wl_p[0], z0, dinv)
    p2 = _msgpass_s16(edge_index, x2)
    return _final(p2, wl_p[1], z1, dinv, pts_p)[:N, :NCLS]


# speedup vs baseline: 11.7215x; 1.0629x over previous
import functools

import jax
import jax.numpy as jnp
from jax import lax
from jax.experimental import pallas as pl
from jax.experimental.pallas import tpu as pltpu
from jax.experimental.pallas import tpu_sc as plsc

N = 10000
E = 320000
IN_DIM = 128
D = 129
DP = 144
DS = 160
NP = 10240
NCLS = 70
T = 4
N_LAYERS = 2
STEP = 0.1
VTH = 1.0

NCORES = 2
NSUB = 16
NW = NCORES * NSUB
EPW = E // NW
NSLOT = 5
ROWS_PER_SUB = NP // NSUB

_mesh = plsc.VectorSubcoreMesh(core_axis_name="c", subcore_axis_name="s")


def _make_msgpass(dtype, width, lanes, K, BLK):
    NBLK = EPW // (BLK * K)
    zsteps = width // lanes

    def body(edges_hbm, x_hbm, out_hbm,
             src_s, dst_s, r0, r1, r2, r3, r4, acc_sh,
             g0, g1, g2, g3, g4, s0, s1, s2, s3, s4):
        rows = (r0, r1, r2, r3, r4)
        gs = (g0, g1, g2, g3, g4)
        ss = (s0, s1, s2, s3, s4)
        c = lax.axis_index("c")
        sid = lax.axis_index("s")
        wid = sid * NCORES + c

        def zb(i, _):
            r0[i // zsteps, pl.ds((i % zsteps) * lanes, lanes)] = (
                jnp.zeros((lanes,), dtype))
            return 0
        lax.fori_loop(0, K * zsteps, zb, 0)

        def zcp(j, _):
            pltpu.sync_copy(
                r0, acc_sh.at[pl.ds(sid * ROWS_PER_SUB + j * K, K)])
            return 0
        lax.fori_loop(0, ROWS_PER_SUB // K, zcp, 0)

        plsc.subcore_barrier()

        ROUNDS = BLK // NSLOT

        def blk(b, _):
            base = wid * EPW + b * BLK * K
            pltpu.sync_copy(edges_hbm.at[0, pl.ds(base, BLK * K)], src_s)
            pltpu.sync_copy(edges_hbm.at[1, pl.ds(base, BLK * K)], dst_s)
            for k in range(NSLOT - 1):
                pltpu.async_copy(
                    x_hbm.at[src_s.at[pl.ds(k * K, K)]], rows[k], gs[k])

            def rnd(i, _):
                j = NSLOT * i
                for k in range(NSLOT):
                    km1 = (k - 1) % NSLOT
                    pltpu.make_async_copy(
                        x_hbm.at[src_s.at[pl.ds((j + k) * K, K)]],
                        rows[k], gs[k]).wait()
                    pltpu.async_copy(
                        rows[k], acc_sh.at[dst_s.at[pl.ds((j + k) * K, K)]],
                        ss[k], add=True)
                    if k == 0:
                        @pl.when(i > 0)
                        def _():
                            pltpu.make_async_copy(
                                rows[km1],
                                acc_sh.at[dst_s.at[pl.ds((j - 1) * K, K)]],
                                ss[km1]).wait()
                        pltpu.async_copy(
                            x_hbm.at[src_s.at[
                                pl.ds((j + NSLOT - 1) * K, K)]],
                            rows[km1], gs[km1])
                    else:
                        pltpu.make_async_copy(
                            rows[km1],
                            acc_sh.at[dst_s.at[pl.ds((j + k - 1) * K, K)]],
                            ss[km1]).wait()

                        @pl.when(i < ROUNDS - 1)
                        def _():
                            pltpu.async_copy(
                                x_hbm.at[src_s.at[
                                    pl.ds((j + k + NSLOT - 1) * K, K)]],
                                rows[km1], gs[km1])
                return 0
            lax.fori_loop(0, ROUNDS, rnd, 0)

            pltpu.make_async_copy(
                rows[(BLK - 1) % NSLOT],
                acc_sh.at[dst_s.at[pl.ds((BLK - 1) * K, K)]],
                ss[(BLK - 1) % NSLOT]).wait()
            return 0
        lax.fori_loop(0, NBLK, blk, 0)

        plsc.subcore_barrier()
        pltpu.sync_copy(
            acc_sh.at[pl.ds(sid * ROWS_PER_SUB, ROWS_PER_SUB)],
            out_hbm.at[c, pl.ds(sid * ROWS_PER_SUB, ROWS_PER_SUB)])

    return functools.partial(
        pl.kernel,
        out_type=jax.ShapeDtypeStruct((NCORES, NP, width), dtype),
        mesh=_mesh,
        scratch_types=[
            pltpu.VMEM((BLK * K,), jnp.int32),
            pltpu.VMEM((BLK * K,), jnp.int32),
            pltpu.VMEM((K, width), dtype),
            pltpu.VMEM((K, width), dtype),
            pltpu.VMEM((K, width), dtype),
            pltpu.VMEM((K, width), dtype),
            pltpu.VMEM((K, width), dtype),
            pltpu.VMEM_SHARED((NP, width), dtype),
            pltpu.SemaphoreType.DMA,
            pltpu.SemaphoreType.DMA,
            pltpu.SemaphoreType.DMA,
            pltpu.SemaphoreType.DMA,
            pltpu.SemaphoreType.DMA,
            pltpu.SemaphoreType.DMA,
            pltpu.SemaphoreType.DMA,
            pltpu.SemaphoreType.DMA,
            pltpu.SemaphoreType.DMA,
            pltpu.SemaphoreType.DMA,
        ],
        compiler_params=pltpu.CompilerParams(use_tc_tiling_on_sc=False),
    )(body)


_msgpass_f32 = _make_msgpass(jnp.float32, DP, 16, K=40, BLK=50)
_msgpass_s16 = _make_msgpass(jnp.int16, DS, 32, K=80, BLK=25)


def _col_ids(shape):
    return lax.broadcasted_iota(jnp.int32, shape, len(shape) - 1)


def _if_dynamics(agg):
    v = jnp.zeros_like(agg)
    ssum = jnp.zeros_like(agg)
    for _ in range(T):
        v = v + agg
        spk = (v >= VTH).astype(jnp.float32)
        v = v - spk * VTH
        ssum = ssum + spk
    return ssum


def _projx(t):
    ids = _col_ids(t.shape)
    rest = jnp.where((ids >= 1) & (ids < D), t, 0.0)
    rss = jnp.sum(rest * rest, axis=-1, keepdims=True)
    x0 = jnp.sqrt(1.0 + rss)
    return jnp.where(ids == 0, x0, rest)


def _to_s16_padded(xs):
    if xs.shape[1] == DS:
        return xs.astype(jnp.int16)
    return jnp.concatenate(
        [xs.astype(jnp.int16),
         jnp.zeros((xs.shape[0], DS - xs.shape[1]), jnp.int16)], axis=1)


NB = 8
BR = NP // NB


def _encode_body(f_ref, w_ref, out_ref):
    mm = jnp.dot(f_ref[...], w_ref[...], preferred_element_type=jnp.float32)
    out_ref[...] = mm + (_col_ids(mm.shape) == D).astype(jnp.float32)


_encode = pl.pallas_call(
    _encode_body,
    grid=(NB,),
    in_specs=[
        pl.BlockSpec((BR, IN_DIM), lambda i: (i, 0)),
        pl.BlockSpec((IN_DIM, DP), lambda i: (0, 0)),
    ],
    out_specs=pl.BlockSpec((BR, DP), lambda i: (i, 0)),
    out_shape=jax.ShapeDtypeStruct((NP, DP), jnp.float32),
)


def _stage0_body(p_ref, x_ref, z_ref, dinv_ref):
    ssum = p_ref[0] + p_ref[1]
    deg = ssum[:, D:D + 1]
    dinv = 1.0 / jnp.maximum(deg, 1.0)
    agg = ssum * dinv
    xs = _if_dynamics(agg)
    xs = jnp.where(_col_ids(xs.shape) < D, xs, 0.0)
    x_ref[...] = _to_s16_padded(xs)
    z_ref[...] = _projx(xs * STEP)
    dinv_ref[...] = dinv


_stage0 = pl.pallas_call(
    _stage0_body,
    grid=(NB,),
    in_specs=[
        pl.BlockSpec((2, BR, DP), lambda i: (0, i, 0)),
    ],
    out_specs=(
        pl.BlockSpec((BR, DS), lambda i: (i, 0)),
        pl.BlockSpec((BR, DP), lambda i: (i, 0)),
        pl.BlockSpec((BR, 1), lambda i: (i, 0)),
    ),
    out_shape=(
        jax.ShapeDtypeStruct((NP, DS), jnp.int16),
        jax.ShapeDtypeStruct((NP, DP), jnp.float32),
        jax.ShapeDtypeStruct((NP, 1), jnp.float32),
    ),
)


def _layer_agg(p_ref, w_ref, dinv_ref):
    m = (p_ref[0] + p_ref[1]).astype(jnp.float32)
    mm = jnp.dot(m, w_ref[...], preferred_element_type=jnp.float32)
    agg = mm * dinv_ref[...]
    xs = _if_dynamics(agg)
    return jnp.where(_col_ids(xs.shape) < D, xs, 0.0)


def _stage1_body(p_ref, w_ref, z_in_ref, dinv_ref, x_ref, z_ref):
    xs = _layer_agg(p_ref, w_ref, dinv_ref)
    x_ref[...] = _to_s16_padded(xs)
    z_ref[...] = _projx(z_in_ref[...] + xs[:, :DP] * STEP)


_stage1 = pl.pallas_call(
    _stage1_body,
    grid=(NB,),
    in_specs=[
        pl.BlockSpec((2, BR, DS), lambda i: (0, i, 0)),
        pl.BlockSpec((DS, DS), lambda i: (0, 0)),
        pl.BlockSpec((BR, DP), lambda i: (i, 0)),
        pl.BlockSpec((BR, 1), lambda i: (i, 0)),
    ],
    out_specs=(
        pl.BlockSpec((BR, DS), lambda i: (i, 0)),
        pl.BlockSpec((BR, DP), lambda i: (i, 0)),
    ),
    out_shape=(
        jax.ShapeDtypeStruct((NP, DS), jnp.int16),
        jax.ShapeDtypeStruct((NP, DP), jnp.float32),
    ),
)


def _final_body(p_ref, w_ref, z_in_ref, dinv_ref, pts_ref, out_ref):
    xs = _layer_agg(p_ref, w_ref, dinv_ref)
    z = _projx(z_in_ref[...] + xs[:, :DP] * STEP)
    q = _projx(pts_ref[...])
    ids = _col_ids(z.shape)
    zneg = jnp.where(ids == 0, -z, z)
    inner = lax.dot_general(zneg, q, (((1,), (1,)), ((), ())),
                            preferred_element_type=jnp.float32)
    c = jnp.maximum(-inner, 1.0 + 1e-5)
    out_ref[...] = -jnp.log(c + jnp.sqrt((c - 1.0) * (c + 1.0)))


_final = pl.pallas_call(
    _final_body,
    grid=(NB,),
    in_specs=[
        pl.BlockSpec((2, BR, DS), lambda i: (0, i, 0)),
        pl.BlockSpec((DS, DS), lambda i: (0, 0)),
        pl.BlockSpec((BR, DP), lambda i: (i, 0)),
        pl.BlockSpec((BR, 1), lambda i: (i, 0)),
        pl.BlockSpec((128, DP), lambda i: (0, 0)),
    ],
    out_specs=pl.BlockSpec((BR, 128), lambda i: (i, 0)),
    out_shape=jax.ShapeDtypeStruct((NP, 128), jnp.float32),
)


def kernel(features, edge_index, W_enc, W_layers, points):
    f_p = jnp.zeros((NP, IN_DIM), jnp.float32).at[:N].set(features)
    we_p = jnp.zeros((IN_DIM, DP), jnp.float32).at[:, :D].set(W_enc)
    wl_p = jnp.zeros((N_LAYERS, DS, DS), jnp.float32).at[:, :D, :D].set(W_layers)
    pts_p = jnp.zeros((128, DP), jnp.float32).at[:NCLS, :D].set(points)

    x0 = _encode(f_p, we_p)
    p0 = _msgpass_f32(edge_index, x0)
    x1, z0, dinv = _stage0(p0)
    p1 = _msgpass_s16(edge_index, x1)
    x2, z1 = _stage1(p1, wl_p[0], z0, dinv)
    p2 = _msgpass_s16(edge_index, x2)
    return _final(p2, wl_p[1], z1, dinv, pts_p)[:N, :NCLS]
